# baseline (speedup 1.0000x reference)
import functools

import jax
import jax.numpy as jnp
from jax import lax
from jax.experimental import pallas as pl
from jax.experimental.pallas import tpu as pltpu
from jax.experimental.pallas import tpu_sc as plsc

_B = 4096
_D = 64
_C_PAD = 128
_NW = 32
_BPW = _B // _NW


def _dpos_sc_body(feats_hbm, labels_hbm, centers_hbm, out_hbm,
                  idx_v, f_v, rows_v, out_v, sem):
    wid = lax.axis_index("s") * 2 + lax.axis_index("c")
    base = wid * _BPW
    pltpu.sync_copy(labels_hbm.at[pl.ds(base, _BPW)], idx_v)
    pltpu.sync_copy(feats_hbm.at[pl.ds(base, _BPW)], f_v)
    pltpu.async_copy(centers_hbm.at[idx_v], rows_v, sem).wait()

    def body(i, _):
        acc = jnp.zeros((16,), jnp.float32)
        for k in range(_D // 16):
            df = f_v[i, pl.ds(k * 16, 16)] - rows_v[i, pl.ds(k * 16, 16)]
            acc = acc + df * df
        out_v[i, :] = acc
        return _

    lax.fori_loop(0, _BPW, body, None)
    pltpu.sync_copy(out_v, out_hbm.at[pl.ds(base, _BPW)])


def _dpos_sc(features, labels, centers):
    k = functools.partial(
        pl.kernel,
        mesh=plsc.VectorSubcoreMesh(core_axis_name="c", subcore_axis_name="s"),
        compiler_params=pltpu.CompilerParams(use_tc_tiling_on_sc=False),
        out_type=jax.ShapeDtypeStruct((_B, 16), jnp.float32),
        scratch_types=[
            pltpu.VMEM((_BPW,), jnp.int32),
            pltpu.VMEM((_BPW, _D), jnp.float32),
            pltpu.VMEM((_BPW, _D), jnp.float32),
            pltpu.VMEM((_BPW, 16), jnp.float32),
            pltpu.SemaphoreType.DMA,
        ],
    )(_dpos_sc_body)
    return k(features, labels, centers)


def _tc_body(n_classes, feats_ref, labels_ref, centers_ref, dpos_ref,
             margin_ref, out_ref):
    f = feats_ref[...]
    c = centers_ref[...]
    labels = labels_ref[...]

    g = jnp.dot(f, c.T, preferred_element_type=jnp.float32)
    fn = jnp.sum(f * f, axis=1, keepdims=True)
    cn = jnp.sum(c * c, axis=1)[None, :]
    d2 = jnp.maximum(fn - 2.0 * g + cn, 0.0)

    col = jax.lax.broadcasted_iota(jnp.int32, d2.shape, 1)
    excl = (col == labels) | (col >= n_classes)
    big = jnp.float32(3.0e38)
    d_neg = jnp.sqrt(jnp.min(jnp.where(excl, big, d2), axis=1))

    d_pos = jnp.sqrt(jnp.sum(dpos_ref[...], axis=1))

    margin = margin_ref[0, 0]
    per_row = jnp.maximum(d_pos + margin - d_neg, 0.0)
    out_ref[0, 0] = jnp.sum(per_row) / _B


def kernel(features, labels, margin, centers):
    n_classes, feat_dim = centers.shape
    centers_p = jnp.zeros((_C_PAD, feat_dim), jnp.float32).at[:n_classes].set(
        centers)
    labels2d = labels.reshape(-1, 1)
    margin_arr = jnp.asarray(margin, jnp.float32).reshape(1, 1)

    dpos_partials = _dpos_sc(features, labels, centers)

    out = pl.pallas_call(
        functools.partial(_tc_body, n_classes),
        out_shape=jax.ShapeDtypeStruct((1, 1), jnp.float32),
        in_specs=[
            pl.BlockSpec(memory_space=pltpu.VMEM),
            pl.BlockSpec(memory_space=pltpu.VMEM),
            pl.BlockSpec(memory_space=pltpu.VMEM),
            pl.BlockSpec(memory_space=pltpu.VMEM),
            pl.BlockSpec(memory_space=pltpu.SMEM),
        ],
        out_specs=pl.BlockSpec(memory_space=pltpu.SMEM),
    )(features, labels2d, centers_p, dpos_partials, margin_arr)
    return out[0, 0]


# speedup vs baseline: 1.0095x; 1.0095x over previous
import functools

import jax
import jax.numpy as jnp
from jax import lax
from jax.experimental import pallas as pl
from jax.experimental.pallas import tpu as pltpu
from jax.experimental.pallas import tpu_sc as plsc

_B = 4096
_D = 64
_C_PAD = 128
_NW = 32
_BPW = _B // _NW


def _dpos_sc_body(feats_hbm, labels_hbm, centers_hbm, out_hbm,
                  idx_v, f_v, rows_v, out_v, sem, fsem):
    wid = lax.axis_index("s") * 2 + lax.axis_index("c")
    base = wid * _BPW
    pltpu.sync_copy(labels_hbm.at[pl.ds(base, _BPW)], idx_v)
    fcopy = pltpu.async_copy(feats_hbm.at[pl.ds(base, _BPW)], f_v, fsem)
    gcopy = pltpu.async_copy(centers_hbm.at[idx_v], rows_v, sem)
    fcopy.wait()
    gcopy.wait()

    for i in range(_BPW):
        acc = None
        for k in range(_D // 16):
            df = f_v[i, pl.ds(k * 16, 16)] - rows_v[i, pl.ds(k * 16, 16)]
            sq = df * df
            acc = sq if acc is None else acc + sq
        out_v[i, :] = acc

    pltpu.sync_copy(out_v, out_hbm.at[pl.ds(base, _BPW)])


def _dpos_sc(features, labels, centers):
    k = functools.partial(
        pl.kernel,
        mesh=plsc.VectorSubcoreMesh(core_axis_name="c", subcore_axis_name="s"),
        compiler_params=pltpu.CompilerParams(use_tc_tiling_on_sc=False),
        out_type=jax.ShapeDtypeStruct((_B, 16), jnp.float32),
        scratch_types=[
            pltpu.VMEM((_BPW,), jnp.int32),
            pltpu.VMEM((_BPW, _D), jnp.float32),
            pltpu.VMEM((_BPW, _D), jnp.float32),
            pltpu.VMEM((_BPW, 16), jnp.float32),
            pltpu.SemaphoreType.DMA,
            pltpu.SemaphoreType.DMA,
        ],
    )(_dpos_sc_body)
    return k(features, labels, centers)


def _dneg_body(n_classes, feats_ref, labels_ref, centers_ref, out_ref):
    f = feats_ref[...]
    c = centers_ref[...]
    labels = labels_ref[...]

    g = jnp.dot(f, c.T, preferred_element_type=jnp.float32)
    fn = jnp.sum(f * f, axis=1, keepdims=True)
    cn = jnp.sum(c * c, axis=1)[None, :]
    d2 = jnp.maximum(fn - 2.0 * g + cn, 0.0)

    col = jax.lax.broadcasted_iota(jnp.int32, d2.shape, 1)
    excl = (col == labels) | (col >= n_classes)
    big = jnp.float32(3.0e38)
    out_ref[...] = jnp.min(jnp.where(excl, big, d2), axis=1, keepdims=True)


def _loss_body(dpos_ref, dneg_ref, margin_ref, out_ref):
    d_pos = jnp.sqrt(jnp.sum(dpos_ref[...], axis=1))
    d_neg = jnp.sqrt(dneg_ref[...][:, 0])
    margin = margin_ref[0, 0]
    per_row = jnp.maximum(d_pos + margin - d_neg, 0.0)
    out_ref[0, 0] = jnp.sum(per_row) / _B


def kernel(features, labels, margin, centers):
    n_classes, feat_dim = centers.shape
    centers_p = jnp.zeros((_C_PAD, feat_dim), jnp.float32).at[:n_classes].set(
        centers)
    labels2d = labels.reshape(-1, 1)
    margin_arr = jnp.asarray(margin, jnp.float32).reshape(1, 1)

    dpos_partials = _dpos_sc(features, labels, centers)

    dneg2 = pl.pallas_call(
        functools.partial(_dneg_body, n_classes),
        out_shape=jax.ShapeDtypeStruct((_B, 1), jnp.float32),
        in_specs=[
            pl.BlockSpec(memory_space=pltpu.VMEM),
            pl.BlockSpec(memory_space=pltpu.VMEM),
            pl.BlockSpec(memory_space=pltpu.VMEM),
        ],
        out_specs=pl.BlockSpec(memory_space=pltpu.VMEM),
    )(features, labels2d, centers_p)

    out = pl.pallas_call(
        _loss_body,
        out_shape=jax.ShapeDtypeStruct((1, 1), jnp.float32),
        in_specs=[
            pl.BlockSpec(memory_space=pltpu.VMEM),
            pl.BlockSpec(memory_space=pltpu.VMEM),
            pl.BlockSpec(memory_space=pltpu.SMEM),
        ],
        out_specs=pl.BlockSpec(memory_space=pltpu.SMEM),
    )(dpos_partials, dneg2, margin_arr)
    return out[0, 0]


# speedup vs baseline: 1.1562x; 1.1453x over previous
import functools

import jax
import jax.numpy as jnp
from jax import lax
from jax.experimental import pallas as pl
from jax.experimental.pallas import tpu as pltpu
from jax.experimental.pallas import tpu_sc as plsc

_B = 4096
_D = 64
_C_PAD = 128
_NW = 32
_BPW = _B // _NW


def _dpos_sc_body(feats_hbm, labels_hbm, centers_hbm, out_hbm,
                  idx_v, f_v, rows_v, acc_v, out_v, sem, fsem):
    wid = lax.axis_index("s") * 2 + lax.axis_index("c")
    base = wid * _BPW
    pltpu.sync_copy(labels_hbm.at[pl.ds(base, _BPW)], idx_v)
    fcopy = pltpu.async_copy(feats_hbm.at[pl.ds(base, _BPW)], f_v, fsem)
    gcopy = pltpu.async_copy(centers_hbm.at[idx_v], rows_v, sem)
    fcopy.wait()
    gcopy.wait()

    lane = lax.iota(jnp.int32, 16)
    for g in range(_BPW // 16):
        for t in range(16):
            i = g * 16 + t
            acc = None
            for k in range(_D // 16):
                df = f_v[i, pl.ds(k * 16, 16)] - rows_v[i, pl.ds(k * 16, 16)]
                sq = df * df
                acc = sq if acc is None else acc + sq
            acc_v[t, :] = acc
        res = None
        for k in range(16):
            col = plsc.load_gather(
                acc_v, [lane, jnp.full((16,), k, jnp.int32)])
            res = col if res is None else res + col
        out_v[pl.ds(g * 16, 16)] = res

    pltpu.sync_copy(out_v, out_hbm.at[pl.ds(base, _BPW)])


def _dpos_sc(features, labels, centers_sc):
    k = functools.partial(
        pl.kernel,
        mesh=plsc.VectorSubcoreMesh(core_axis_name="c", subcore_axis_name="s"),
        compiler_params=pltpu.CompilerParams(needs_layout_passes=False),
        out_type=jax.ShapeDtypeStruct((_B,), jnp.float32),
        scratch_types=[
            pltpu.VMEM((_BPW,), jnp.int32),
            pltpu.VMEM((_BPW, _D), jnp.float32),
            pltpu.VMEM((_BPW, _C_PAD), jnp.float32),
            pltpu.VMEM((16, 16), jnp.float32),
            pltpu.VMEM((_BPW,), jnp.float32),
            pltpu.SemaphoreType.DMA,
            pltpu.SemaphoreType.DMA,
        ],
    )(_dpos_sc_body)
    return k(features, labels, centers_sc)


_R = 112


def _dneg_body(feats_ref, labels_ref, caug_ref, cn_ref, out_ref):
    f = feats_ref[...]
    ft = f.T
    fct = jnp.concatenate([ft, ft * ft], axis=0)

    caug = caug_ref[...]
    g = jnp.dot(caug, fct, preferred_element_type=jnp.float32)
    fn = g[0:1, :]
    cn = cn_ref[...]

    labels = labels_ref[...]
    row = jax.lax.broadcasted_iota(jnp.int32, g.shape, 0)
    valid = (row >= 8) & (row < 108) & (row != labels + 8)
    big = jnp.float32(3.0e38)
    m = jnp.min(jnp.where(valid, cn - 2.0 * g, big), axis=0, keepdims=True)
    out_ref[...] = jnp.maximum(fn + m, 0.0)


def _loss_body(dpos_ref, dneg_ref, margin_ref, out_ref):
    d_pos = jnp.sqrt(dpos_ref[...])
    d_neg = jnp.sqrt(dneg_ref[...])
    margin = margin_ref[0, 0]
    per_row = jnp.maximum(d_pos + margin - d_neg, 0.0)
    out_ref[0, 0] = jnp.sum(per_row) / _B


def kernel(features, labels, margin, centers):
    n_classes, feat_dim = centers.shape
    centers_sc = jnp.zeros((n_classes, _C_PAD), jnp.float32).at[:, :feat_dim
                                                                ].set(centers)
    caug = (jnp.zeros((_R, 2 * feat_dim), jnp.float32)
            .at[0, feat_dim:].set(1.0)
            .at[8:8 + n_classes, :feat_dim].set(centers))
    cn = jnp.sum(centers * centers, axis=1)
    cn_aug = jnp.zeros((_R, 1), jnp.float32).at[8:8 + n_classes, 0].set(cn)
    labels_lane = labels.reshape(1, _B)
    margin_arr = jnp.asarray(margin, jnp.float32).reshape(1, 1)

    dpos2 = _dpos_sc(features, labels, centers_sc)

    dneg2 = pl.pallas_call(
        _dneg_body,
        out_shape=jax.ShapeDtypeStruct((1, _B), jnp.float32),
        in_specs=[
            pl.BlockSpec(memory_space=pltpu.VMEM),
            pl.BlockSpec(memory_space=pltpu.VMEM),
            pl.BlockSpec(memory_space=pltpu.VMEM),
            pl.BlockSpec(memory_space=pltpu.VMEM),
        ],
        out_specs=pl.BlockSpec(memory_space=pltpu.VMEM),
    )(features, labels_lane, caug, cn_aug)

    out = pl.pallas_call(
        _loss_body,
        out_shape=jax.ShapeDtypeStruct((1, 1), jnp.float32),
        in_specs=[
            pl.BlockSpec(memory_space=pltpu.VMEM),
            pl.BlockSpec(memory_space=pltpu.VMEM),
            pl.BlockSpec(memory_space=pltpu.SMEM),
        ],
        out_specs=pl.BlockSpec(memory_space=pltpu.SMEM),
    )(dpos2.reshape(1, _B), dneg2, margin_arr)
    return out[0, 0]
